# baseline (speedup 1.0000x reference)
import functools

import jax
import jax.numpy as jnp
from jax import lax
from jax.experimental import pallas as pl
from jax.experimental.pallas import tpu as pltpu
from jax.experimental.pallas import tpu_sc as plsc

B = 4096
L = 200
EMB = 32
VOCAB = 1000000

NC = 2
NS = 16
NW = NC * NS
BPW = B // NW
IPW = BPW * L

_CH = 256
_NCH = 120
_RPW = _CH * _NCH
_SC_ROWS = NW * _RPW

_MV_BLK = 16384
_SC_BLKS = _SC_ROWS // _MV_BLK
_MV_NBLK = -(-(VOCAB - _SC_ROWS) // _MV_BLK)
_TV_LEN = _SC_ROWS + _MV_NBLK * _MV_BLK


def _mv_body(wr_ref, a_ref, bl_ref, o_ref):
    r = lax.dot_general(
        wr_ref[...], a_ref[...], (((1,), (1,)), ((), ())),
        preferred_element_type=jnp.float32,
    )
    o_ref[...] = (r + bl_ref[0, 0]).reshape(_MV_BLK)


def _tc_matvec(table, wr, bl):
    return pl.pallas_call(
        _mv_body,
        grid=(_MV_NBLK,),
        in_specs=[
            pl.BlockSpec((1, EMB), lambda i: (0, 0)),
            pl.BlockSpec((_MV_BLK, EMB), lambda i: (i + _SC_BLKS, 0)),
            pl.BlockSpec(memory_space=pltpu.SMEM),
        ],
        out_specs=pl.BlockSpec((_MV_BLK,), lambda i: (i,)),
        out_shape=jax.ShapeDtypeStruct((_MV_NBLK * _MV_BLK,), jnp.float32),
    )(wr, table, bl)


_mesh = plsc.VectorSubcoreMesh(core_axis_name="c", subcore_axis_name="s")


@functools.partial(
    pl.kernel,
    mesh=_mesh,
    out_type=jax.ShapeDtypeStruct((_SC_ROWS,), jnp.float32),
    compiler_params=pltpu.CompilerParams(needs_layout_passes=False),
    scratch_types=[
        pltpu.VMEM((_CH, EMB), jnp.float32),
        pltpu.VMEM((_CH, EMB), jnp.float32),
        pltpu.VMEM((1, EMB), jnp.float32),
        pltpu.VMEM((16,), jnp.float32),
        pltpu.VMEM((256,), jnp.float32),
        pltpu.VMEM((_CH,), jnp.float32),
        pltpu.SemaphoreType.DMA,
        pltpu.SemaphoreType.DMA,
    ],
)
def _sc_matvec(tab_hbm, wr_hbm, bl_hbm, tv_hbm,
               buf0, buf1, w_v, bl_v, pmem, tvbuf, sem0, sem1):
    wid = lax.axis_index("s") * NC + lax.axis_index("c")
    base = wid * _RPW

    pltpu.sync_copy(wr_hbm, w_v)
    pltpu.sync_copy(bl_hbm, bl_v)
    wlo = w_v[0, pl.ds(0, 16)]
    whi = w_v[0, pl.ds(16, 16)]
    blv = bl_v[...]
    lane16 = lax.iota(jnp.int32, 16) * 16

    def start(c, buf, sem):
        pltpu.async_copy(tab_hbm.at[pl.ds(base + c * _CH, _CH), :], buf, sem)

    def wait(c, buf, sem):
        pltpu.make_async_copy(
            tab_hbm.at[pl.ds(base + c * _CH, _CH), :], buf, sem).wait()

    def compute(c, buf):
        def group(g, _):
            r0 = g * 16
            for j in range(16):
                s = buf[r0 + j, pl.ds(0, 16)] * wlo + buf[r0 + j, pl.ds(16, 16)] * whi
                pmem[pl.ds(j * 16, 16)] = s
            acc = blv
            for k in range(16):
                acc = acc + plsc.load_gather(pmem, [lane16 + k])
            tvbuf[pl.ds(r0, 16)] = acc
            return 0

        lax.fori_loop(0, _CH // 16, group, 0)
        pltpu.sync_copy(tvbuf, tv_hbm.at[pl.ds(base + c * _CH, _CH)])

    start(0, buf0, sem0)
    start(1, buf1, sem1)

    def pair(p, _):
        c0 = p * 2
        wait(c0, buf0, sem0)
        compute(c0, buf0)

        @pl.when(p + 1 < _NCH // 2)
        def _():
            start(c0 + 2, buf0, sem0)

        wait(c0 + 1, buf1, sem1)
        compute(c0 + 1, buf1)

        @pl.when(p + 1 < _NCH // 2)
        def _():
            start(c0 + 3, buf1, sem1)
        return 0

    lax.fori_loop(0, _NCH // 2, pair, 0)


@functools.partial(
    pl.kernel,
    mesh=_mesh,
    out_type=jax.ShapeDtypeStruct((B,), jnp.float32),
    compiler_params=pltpu.CompilerParams(needs_layout_passes=False),
    scratch_types=[
        pltpu.VMEM((IPW,), jnp.int32),
        pltpu.VMEM((IPW,), jnp.float32),
        pltpu.VMEM((BPW,), jnp.float32),
        pltpu.SemaphoreType.DMA,
    ],
)
def _sc_pool(ids_hbm, tv_hbm, out_hbm, idx_v, vals_v, out_v, sem):
    wid = lax.axis_index("s") * NC + lax.axis_index("c")
    pltpu.sync_copy(ids_hbm.at[pl.ds(wid * IPW, IPW)], idx_v)
    pltpu.async_copy(tv_hbm.at[idx_v], vals_v, sem).wait()

    lanebase = lax.iota(jnp.int32, 16) * L
    for c in range(BPW // 16):
        def body(l, acc):
            return acc + plsc.load_gather(vals_v, [lanebase + (c * 16 * L + l)])

        acc = lax.fori_loop(0, L, body, jnp.zeros((16,), jnp.float32))
        out_v[pl.ds(c * 16, 16)] = 1.0 / (1.0 + jnp.exp(-acc))
    pltpu.sync_copy(out_v, out_hbm.at[pl.ds(wid * BPW, BPW)])


def kernel(input_ids, table, W, b):
    wr = (W * (1.0 / L)).astype(jnp.float32)
    bl = (b * (1.0 / L)).reshape(1, 1).astype(jnp.float32)
    bl16 = jnp.broadcast_to(bl.reshape(()), (16,))

    tv_sc = _sc_matvec(table, wr, bl16)
    tv_tc = _tc_matvec(table, wr, bl)
    tv = jnp.concatenate([tv_sc, tv_tc])
    ids_flat = input_ids.reshape(NW * IPW)
    out = _sc_pool(ids_flat, tv)
    return out.reshape(B, 1)


# speedup vs baseline: 1.2390x; 1.2390x over previous
import functools

import jax
import jax.numpy as jnp
from jax import lax
from jax.experimental import pallas as pl
from jax.experimental.pallas import tpu as pltpu
from jax.experimental.pallas import tpu_sc as plsc

B = 4096
L = 200
EMB = 32
VOCAB = 1000000

NC = 2
NS = 16
NW = NC * NS
BPW = B // NW
IPW = BPW * L

_MV_BLK = 49152
_MV_NBLK = -(-VOCAB // _MV_BLK)
_TV_LEN = _MV_NBLK * _MV_BLK


def _mv_body(wr_ref, a_ref, bl_ref, o_ref):
    r = lax.dot_general(
        wr_ref[...], a_ref[...], (((1,), (1,)), ((), ())),
        preferred_element_type=jnp.float32,
    )
    o_ref[...] = (r + bl_ref[0, 0]).reshape(_MV_BLK)


def _tc_matvec(table, wr, bl):
    return pl.pallas_call(
        _mv_body,
        grid=(_MV_NBLK,),
        in_specs=[
            pl.BlockSpec((1, EMB), lambda i: (0, 0)),
            pl.BlockSpec((_MV_BLK, EMB), lambda i: (i, 0)),
            pl.BlockSpec(memory_space=pltpu.SMEM),
        ],
        out_specs=pl.BlockSpec((_MV_BLK,), lambda i: (i,)),
        out_shape=jax.ShapeDtypeStruct((_TV_LEN,), jnp.float32),
    )(wr, table, bl)


_mesh = plsc.VectorSubcoreMesh(core_axis_name="c", subcore_axis_name="s")


@functools.partial(
    pl.kernel,
    mesh=_mesh,
    out_type=jax.ShapeDtypeStruct((B,), jnp.float32),
    compiler_params=pltpu.CompilerParams(needs_layout_passes=False),
    scratch_types=[
        pltpu.VMEM((IPW,), jnp.int32),
        pltpu.VMEM((IPW,), jnp.float32),
        pltpu.VMEM((BPW,), jnp.float32),
        pltpu.SemaphoreType.DMA,
    ],
)
def _sc_pool(ids_hbm, tv_hbm, out_hbm, idx_v, vals_v, out_v, sem):
    wid = lax.axis_index("s") * NC + lax.axis_index("c")
    pltpu.sync_copy(ids_hbm.at[pl.ds(wid * IPW, IPW)], idx_v)
    pltpu.async_copy(tv_hbm.at[idx_v], vals_v, sem).wait()

    lanebase = lax.iota(jnp.int32, 16) * L
    for c in range(BPW // 16):
        def body(l, acc):
            return acc + plsc.load_gather(vals_v, [lanebase + (c * 16 * L + l)])

        acc = lax.fori_loop(0, L, body, jnp.zeros((16,), jnp.float32))
        out_v[pl.ds(c * 16, 16)] = 1.0 / (1.0 + jnp.exp(-acc))
    pltpu.sync_copy(out_v, out_hbm.at[pl.ds(wid * BPW, BPW)])


def kernel(input_ids, table, W, b):
    wr = (W * (1.0 / L)).astype(jnp.float32)
    bl = (b * (1.0 / L)).reshape(1, 1).astype(jnp.float32)

    tv = _tc_matvec(table, wr, bl)
    ids_flat = input_ids.reshape(NW * IPW)
    out = _sc_pool(ids_flat, tv)
    return out.reshape(B, 1)


# speedup vs baseline: 1.2411x; 1.0017x over previous
import functools

import jax
import jax.numpy as jnp
from jax import lax
from jax.experimental import pallas as pl
from jax.experimental.pallas import tpu as pltpu
from jax.experimental.pallas import tpu_sc as plsc

B = 4096
L = 200
EMB = 32
VOCAB = 1000000

NC = 2
NS = 16
NW = NC * NS
BPW = B // NW
IPW = BPW * L

_MV_BLK = 16384
_MV_NBLK = -(-VOCAB // _MV_BLK)
_TV_LEN = _MV_NBLK * _MV_BLK


def _mv_body(wr_ref, a_ref, bl_ref, o_ref):
    r = lax.dot_general(
        wr_ref[...], a_ref[...], (((1,), (1,)), ((), ())),
        preferred_element_type=jnp.float32,
    )
    o_ref[...] = (r + bl_ref[0, 0]).reshape(_MV_BLK)


def _tc_matvec(table, wr, bl):
    return pl.pallas_call(
        _mv_body,
        grid=(_MV_NBLK,),
        in_specs=[
            pl.BlockSpec((1, EMB), lambda i: (0, 0)),
            pl.BlockSpec((_MV_BLK, EMB), lambda i: (i, 0)),
            pl.BlockSpec(memory_space=pltpu.SMEM),
        ],
        out_specs=pl.BlockSpec((_MV_BLK,), lambda i: (i,)),
        out_shape=jax.ShapeDtypeStruct((_TV_LEN,), jnp.float32),
    )(wr, table, bl)


_mesh = plsc.VectorSubcoreMesh(core_axis_name="c", subcore_axis_name="s")


@functools.partial(
    pl.kernel,
    mesh=_mesh,
    out_type=jax.ShapeDtypeStruct((B,), jnp.float32),
    compiler_params=pltpu.CompilerParams(needs_layout_passes=False),
    scratch_types=[
        pltpu.VMEM((IPW // 2,), jnp.int32),
        pltpu.VMEM((IPW // 2,), jnp.int32),
        pltpu.VMEM((IPW // 2,), jnp.float32),
        pltpu.VMEM((IPW // 2,), jnp.float32),
        pltpu.VMEM((BPW,), jnp.float32),
        pltpu.SemaphoreType.DMA,
        pltpu.SemaphoreType.DMA,
    ],
)
def _sc_pool(ids_hbm, tv_hbm, out_hbm, idx0, idx1, vals0, vals1, out_v,
             sem0, sem1):
    wid = lax.axis_index("s") * NC + lax.axis_index("c")
    pltpu.sync_copy(ids_hbm.at[pl.ds(wid * IPW, IPW // 2)], idx0)
    pltpu.async_copy(tv_hbm.at[idx0], vals0, sem0)
    pltpu.sync_copy(ids_hbm.at[pl.ds(wid * IPW + IPW // 2, IPW // 2)], idx1)
    pltpu.async_copy(tv_hbm.at[idx1], vals1, sem1)

    lanebase = lax.iota(jnp.int32, 16) * L
    for h, (vals, sem) in enumerate(((vals0, sem0), (vals1, sem1))):
        pltpu.make_async_copy(tv_hbm.at[idx0], vals, sem).wait()
        for c in range(BPW // 32):
            def body(l, acc):
                return acc + plsc.load_gather(
                    vals, [lanebase + (c * 16 * L + l)])

            acc = lax.fori_loop(0, L, body, jnp.zeros((16,), jnp.float32))
            out_v[pl.ds(h * (BPW // 2) + c * 16, 16)] = 1.0 / (1.0 + jnp.exp(-acc))
    pltpu.sync_copy(out_v, out_hbm.at[pl.ds(wid * BPW, BPW)])


def kernel(input_ids, table, W, b):
    wr = (W * (1.0 / L)).astype(jnp.float32)
    bl = (b * (1.0 / L)).reshape(1, 1).astype(jnp.float32)

    tv = _tc_matvec(table, wr, bl)
    ids_flat = input_ids.reshape(NW * IPW)
    out = _sc_pool(ids_flat, tv)
    return out.reshape(B, 1)


# speedup vs baseline: 1.2422x; 1.0009x over previous
import functools

import jax
import jax.numpy as jnp
from jax import lax
from jax.experimental import pallas as pl
from jax.experimental.pallas import tpu as pltpu
from jax.experimental.pallas import tpu_sc as plsc

B = 4096
L = 200
EMB = 32
VOCAB = 1000000

NC = 2
NS = 16
NW = NC * NS
BPW = B // NW
IPW = BPW * L

_MV_BLK = 16384
_MV_NBLK = -(-VOCAB // _MV_BLK)
_TV_LEN = _MV_NBLK * _MV_BLK


def _mv_body(wr_ref, a_ref, bl_ref, o_ref):
    r = lax.dot_general(
        wr_ref[...], a_ref[...], (((1,), (1,)), ((), ())),
        preferred_element_type=jnp.float32,
    )
    o_ref[...] = (r + bl_ref[0, 0]).reshape(_MV_BLK)


def _tc_matvec(table, wr, bl):
    return pl.pallas_call(
        _mv_body,
        grid=(_MV_NBLK,),
        in_specs=[
            pl.BlockSpec((1, EMB), lambda i: (0, 0)),
            pl.BlockSpec((_MV_BLK, EMB), lambda i: (i, 0)),
            pl.BlockSpec(memory_space=pltpu.SMEM),
        ],
        out_specs=pl.BlockSpec((_MV_BLK,), lambda i: (i,)),
        out_shape=jax.ShapeDtypeStruct((_TV_LEN,), jnp.float32),
    )(wr, table, bl)


_mesh = plsc.VectorSubcoreMesh(core_axis_name="c", subcore_axis_name="s")


@functools.partial(
    pl.kernel,
    mesh=_mesh,
    out_type=jax.ShapeDtypeStruct((B,), jnp.float32),
    compiler_params=pltpu.CompilerParams(needs_layout_passes=False),
    scratch_types=[
        pltpu.VMEM((IPW,), jnp.int32),
        pltpu.VMEM((IPW,), jnp.float32),
        pltpu.VMEM((BPW,), jnp.float32),
        pltpu.SemaphoreType.DMA,
    ],
)
def _sc_pool(ids_hbm, tv_hbm, out_hbm, idx_v, vals_v, out_v, sem):
    wid = lax.axis_index("s") * NC + lax.axis_index("c")
    pltpu.sync_copy(ids_hbm.at[pl.ds(wid * IPW, IPW)], idx_v)
    pltpu.async_copy(tv_hbm.at[idx_v], vals_v, sem).wait()

    lanebase = lax.iota(jnp.int32, 16) * L
    for c in range(BPW // 16):
        def body(l, acc):
            return acc + plsc.load_gather(vals_v, [lanebase + (c * 16 * L + l)])

        acc = lax.fori_loop(0, L, body, jnp.zeros((16,), jnp.float32))
        out_v[pl.ds(c * 16, 16)] = 1.0 / (1.0 + jnp.exp(-acc))
    pltpu.sync_copy(out_v, out_hbm.at[pl.ds(wid * BPW, BPW)])


def kernel(input_ids, table, W, b):
    wr = (W * (1.0 / L)).astype(jnp.float32)
    bl = (b * (1.0 / L)).reshape(1, 1).astype(jnp.float32)

    tv = _tc_matvec(table, wr, bl)
    ids_flat = input_ids.reshape(NW * IPW)
    out = _sc_pool(ids_flat, tv)
    return out.reshape(B, 1)
